# baseline (speedup 1.0000x reference)
import functools

import jax
import jax.numpy as jnp
from jax import lax
from jax.experimental import pallas as pl
from jax.experimental.pallas import tpu as pltpu
from jax.experimental.pallas import tpu_sc as plsc

N = 9996
E = 319872
NG = 833
DIN = 128
H = 64

NC = 2
NS = 16
NW = NC * NS
L = 16

EPW = 10240
EP = EPW * NW
NCH = 80
CH = 128
AR = 10240
NPB = 10240
GPB = 848
GF = GPB * H

_HI = jax.lax.Precision.HIGHEST


def _iota16():
  return lax.iota(jnp.int32, L)


def _sc_hist_body(dst_rs, batch_rs, deg_out, cnt_out, idx, ones_v, bidx, zb,
                  degacc, cntacc):
  s = lax.axis_index("s")
  c = lax.axis_index("c")
  w = s * NC + c

  one = jnp.ones((L,), jnp.int32)
  zero = jnp.zeros((L,), jnp.int32)
  for j in range(CH // L):
    ones_v[pl.ds(j * L, L)] = one
  for j in range(1024 // L):
    zb[pl.ds(j * L, L)] = zero

  pltpu.sync_copy(zb.at[pl.ds(0, 640)], degacc.at[pl.ds(s * 640, 640)])

  @pl.when(s == 0)
  def _():
    pltpu.sync_copy(zb.at[pl.ds(0, 848)], cntacc)

  plsc.subcore_barrier()

  pltpu.sync_copy(dst_rs.at[w], idx)

  def _chunk(jo, _):
    for ji in range(4):
      pltpu.sync_copy(ones_v, degacc.at[idx.at[jo * 4 + ji]], add=True)
    return 0

  lax.fori_loop(0, NCH // 4, _chunk, 0)

  pltpu.sync_copy(batch_rs.at[w], bidx)
  for j in range(4):
    pltpu.sync_copy(ones_v.at[pl.ds(0, 80)], cntacc.at[bidx.at[j]], add=True)

  plsc.subcore_barrier()

  pltpu.sync_copy(degacc.at[pl.ds(s * 640, 640)],
                  deg_out.at[c, pl.ds(s * 640, 640)])

  @pl.when(s == 0)
  def _():
    pltpu.sync_copy(cntacc, cnt_out.at[c])


def _sc_hist(dst_rs, batch_rs):
  f = pl.kernel(
      _sc_hist_body,
      out_type=(
          jax.ShapeDtypeStruct((NC, AR), jnp.int32),
          jax.ShapeDtypeStruct((NC, GPB), jnp.int32),
      ),
      mesh=plsc.VectorSubcoreMesh(core_axis_name="c", subcore_axis_name="s",
                                  num_cores=NC, num_subcores=NS),
      scratch_types=[
          pltpu.VMEM((NCH, CH), jnp.int32),
          pltpu.VMEM((CH,), jnp.int32),
          pltpu.VMEM((4, 80), jnp.int32),
          pltpu.VMEM((1024,), jnp.int32),
          pltpu.VMEM_SHARED((AR,), jnp.int32),
          pltpu.VMEM_SHARED((GPB,), jnp.int32),
      ],
  )
  return f(dst_rs, batch_rs)


def _sc_scatter_body(t_hbm, src_rs, dst_rs, out_hbm, idxs, idxd, rows, rows2,
                     rows3, rows4, rows5, rows6, rows7, rows8, zb, acc, sem,
                     sem2):
  s = lax.axis_index("s")
  c = lax.axis_index("c")
  w = s * NC + c

  zrow = jnp.zeros((L,), jnp.float32)
  for r in range(64):
    for k in range(H // L):
      zb[r, pl.ds(k * L, L)] = zrow

  for i in range(10):
    pltpu.sync_copy(zb, acc.at[pl.ds(s * 640 + i * 64, 64)])

  plsc.subcore_barrier()

  pltpu.sync_copy(src_rs.at[w], idxs)
  pltpu.sync_copy(dst_rs.at[w], idxd)

  bufs = (rows, rows2, rows3, rows4, rows5, rows6, rows7, rows8)
  for j in range(7):
    pltpu.make_async_copy(t_hbm.at[idxs.at[j]], bufs[j], sem).start()

  def _chunk(jo, _):
    for ji in range(8):
      j = jo * 8 + ji
      bufa = bufs[ji]
      bufb = bufs[(ji + 7) % 8]
      pltpu.make_async_copy(
          t_hbm.at[idxs.at[jnp.minimum(j, NCH - 1)]], bufa, sem).wait()

      @pl.when(j >= 1)
      def _():
        pltpu.make_async_copy(
            bufb, acc.at[idxd.at[jnp.maximum(j - 1, 0)]], sem2).wait()

      @pl.when(j < NCH - 7)
      def _():
        pltpu.make_async_copy(
            t_hbm.at[idxs.at[jnp.minimum(j + 7, NCH - 1)]], bufb, sem).start()

      pltpu.make_async_copy(bufa, acc.at[idxd.at[j]], sem2).start(add=True)
    return 0

  lax.fori_loop(0, NCH // 8, _chunk, 0)
  pltpu.make_async_copy(bufs[7], acc.at[idxd.at[NCH - 1]], sem2).wait()

  plsc.subcore_barrier()

  pltpu.sync_copy(acc.at[pl.ds(s * 640, 640)],
                  out_hbm.at[c, pl.ds(s * 640, 640)])


def _sc_scatter(t, src_rs, dst_rs):
  f = pl.kernel(
      _sc_scatter_body,
      out_type=jax.ShapeDtypeStruct((NC, AR, H), jnp.float32),
      mesh=plsc.VectorSubcoreMesh(core_axis_name="c", subcore_axis_name="s",
                                  num_cores=NC, num_subcores=NS),
      compiler_params=pltpu.CompilerParams(use_tc_tiling_on_sc=False),
      scratch_types=[
          pltpu.VMEM((NCH, CH), jnp.int32),
          pltpu.VMEM((NCH, CH), jnp.int32),
          pltpu.VMEM((CH, H), jnp.float32),
          pltpu.VMEM((CH, H), jnp.float32),
          pltpu.VMEM((CH, H), jnp.float32),
          pltpu.VMEM((CH, H), jnp.float32),
          pltpu.VMEM((CH, H), jnp.float32),
          pltpu.VMEM((CH, H), jnp.float32),
          pltpu.VMEM((CH, H), jnp.float32),
          pltpu.VMEM((CH, H), jnp.float32),
          pltpu.VMEM((64, H), jnp.float32),
          pltpu.VMEM_SHARED((AR, H), jnp.float32),
          pltpu.SemaphoreType.DMA,
          pltpu.SemaphoreType.DMA,
      ],
  )
  return f(t, src_rs, dst_rs)


def _sc_pool_body(hf_flat, batch_rs, max_out, sum_out, rowsf, bsm, lmax, lsum,
                  sem):
  s = lax.axis_index("s")
  c = lax.axis_index("c")
  w = s * NC + c

  zrow = jnp.zeros((L,), jnp.float32)

  def _zero(i, _):
    lmax[pl.ds(i * L, L)] = zrow
    lsum[pl.ds(i * L, L)] = zrow
    return 0

  lax.fori_loop(0, GF // L, _zero, 0)

  pltpu.sync_copy(hf_flat.at[pl.ds(w * 320 * H, 320 * H)], rowsf)
  pltpu.sync_copy(batch_rs.at[w], bsm)

  it = _iota16()

  def _row(r, _):
    g = bsm[pl.ds(r, L)][0]
    base = g * H
    rbase = r * H
    for k in range(H // L):
      fidx = base + k * L + it
      v = plsc.load_gather(rowsf, [rbase + k * L + it])
      cur = plsc.load_gather(lmax, [fidx])
      plsc.store_scatter(lmax, [fidx], jnp.maximum(cur, v))
      plsc.addupdate_scatter(lsum, [fidx], v)
    return 0

  lax.fori_loop(0, 320, _row, 0)

  pltpu.sync_copy(lmax, max_out.at[w])
  pltpu.sync_copy(lsum, sum_out.at[w])


def _sc_pool(hf_flat, batch_rs):
  f = pl.kernel(
      _sc_pool_body,
      out_type=(
          jax.ShapeDtypeStruct((NW, GF), jnp.float32),
          jax.ShapeDtypeStruct((NW, GF), jnp.float32),
      ),
      mesh=plsc.VectorSubcoreMesh(core_axis_name="c", subcore_axis_name="s",
                                  num_cores=NC, num_subcores=NS),
      compiler_params=pltpu.CompilerParams(needs_layout_passes=False),
      scratch_types=[
          pltpu.VMEM((320 * H,), jnp.float32),
          pltpu.VMEM((384,), jnp.int32),
          pltpu.VMEM((GF,), jnp.float32),
          pltpu.VMEM((GF,), jnp.float32),
          pltpu.SemaphoreType.DMA,
      ],
  )
  return f(hf_flat, batch_rs)


def _tc_a_body(dp_t, x, w1, t1, dinv):
  deg = (dp_t[:, 0:1] + dp_t[:, 1:2] + 1).astype(jnp.float32)
  dv = lax.rsqrt(deg)
  xw = lax.dot_general(x[...], w1[...], (((1,), (1,)), ((), ())), precision=_HI,
                       preferred_element_type=jnp.float32)
  t1[...] = xw * dv[:N]
  dinv[...] = dv[:N]


def _tc_a(dp_t, x, w1):
  return pl.pallas_call(
      _tc_a_body,
      out_shape=(
          jax.ShapeDtypeStruct((N, H), jnp.float32),
          jax.ShapeDtypeStruct((N, 1), jnp.float32),
      ),
  )(dp_t, x, w1)


def _tc_b_body(sp, t, dinv, b, wn, tn):
  agg = (sp[0, :N] + sp[1, :N] + t[...]) * dinv[...] + b[...]
  h = jnp.maximum(agg, 0.0)
  hw = lax.dot_general(h, wn[...], (((1,), (1,)), ((), ())), precision=_HI,
                       preferred_element_type=jnp.float32)
  tn[...] = hw * dinv[...]


def _tc_b(sp, t, dinv, b2d, wn):
  return pl.pallas_call(
      _tc_b_body,
      out_shape=jax.ShapeDtypeStruct((N, H), jnp.float32),
  )(sp, t, dinv, b2d, wn)


def _tc_c1_body(sp, t, dinv, b, wattn, wv, h_out, att_out):
  agg = (sp[0, :N] + sp[1, :N] + t[...]) * dinv[...] + b[...]
  h = jnp.maximum(agg, 0.0)
  energy = jnp.tanh(lax.dot_general(h, wattn[...], (((1,), (1,)), ((), ())),
                                    precision=_HI,
                                    preferred_element_type=jnp.float32))
  att = lax.dot_general(energy, wv[...], (((1,), (1,)), ((), ())),
                        precision=_HI, preferred_element_type=jnp.float32)
  h_out[...] = h
  att_out[...] = att


def _tc_c1(sp, t, dinv, b2d, wattn, wv):
  return pl.pallas_call(
      _tc_c1_body,
      out_shape=(
          jax.ShapeDtypeStruct((N, H), jnp.float32),
          jax.ShapeDtypeStruct((N, 1), jnp.float32),
      ),
  )(sp, t, dinv, b2d, wattn, wv)


def _tc_c2_body(att, h_rs, hf_out, sc_out):
  a = att[...]
  m = jnp.max(a, axis=1, keepdims=True)
  ex = jnp.exp(a - m)
  sc = ex / jnp.sum(ex, axis=1, keepdims=True)
  rows = lax.broadcasted_iota(jnp.int32, (12, 12 * H), 0)
  cols = lax.broadcasted_iota(jnp.int32, (12, 12 * H), 1) // H
  rmat = (rows == cols).astype(jnp.float32)
  expd = lax.dot_general(sc, rmat, (((1,), (0,)), ((), ())), precision=_HI,
                         preferred_element_type=jnp.float32)
  hf_out[...] = h_rs[...] * expd
  sc_out[...] = sc


def _tc_c2(att_rs, h_rs):
  return pl.pallas_call(
      _tc_c2_body,
      out_shape=(
          jax.ShapeDtypeStruct((NG, 12 * H), jnp.float32),
          jax.ShapeDtypeStruct((NG, 12), jnp.float32),
      ),
  )(att_rs, h_rs)


def _tc_d_body(maxp, sump, cnt_t, wlin, blin, out):
  gmax = jnp.max(maxp[...], axis=0)[:NG]
  gsum = jnp.sum(sump[...], axis=0)[:NG]
  cnt = (cnt_t[:, 0:1] + cnt_t[:, 1:2]).astype(jnp.float32)[:NG]
  gmean = gsum / jnp.maximum(cnt, 1.0)
  pooled = jnp.concatenate([gmax, gmean], axis=1)
  ow = lax.dot_general(pooled, wlin[...], (((1,), (1,)), ((), ())),
                       precision=_HI, preferred_element_type=jnp.float32)
  out[...] = jnp.maximum(ow + blin[...], 0.0)


def _tc_d(maxp3, sump3, cnt_t, wlin, blin2d):
  return pl.pallas_call(
      _tc_d_body,
      out_shape=jax.ShapeDtypeStruct((NG, 32), jnp.float32),
  )(maxp3, sump3, cnt_t, wlin, blin2d)


def kernel(data_x, data_edge_index, data_batch, W1, b1, W2, b2, W3, b3, Wattn,
           Wv, Wlin, blin):
  src = data_edge_index[0].astype(jnp.int32)
  dst = data_edge_index[1].astype(jnp.int32)
  npad = EP - E
  pad_ar = lax.iota(jnp.int32, npad)
  src_pad = jnp.concatenate([src, (pad_ar * 1237) % N])
  dst_pad = jnp.concatenate([dst, N + pad_ar % (AR - N)])
  src_rs = src_pad.reshape(NW, NCH, CH)
  dst_rs = dst_pad.reshape(NW, NCH, CH)

  batch_pad = jnp.concatenate([
      data_batch.astype(jnp.int32),
      jnp.full((NPB - N,), NG, jnp.int32),
  ])
  batch_rs = batch_pad.reshape(NW, 4, 80)
  batch_rows = jnp.concatenate(
      [batch_pad.reshape(NW, 320), jnp.full((NW, 64), NG, jnp.int32)], axis=1)

  degp, cntp = _sc_hist(dst_rs, batch_rs)
  dp_t = degp.T
  cnt_t = cntp.T

  b1_2d = b1.reshape(1, H)
  b2_2d = b2.reshape(1, H)
  b3_2d = b3.reshape(1, H)
  blin2d = blin.reshape(1, 32)

  t1, dinv = _tc_a(dp_t, data_x, W1)
  s1 = _sc_scatter(t1, src_rs, dst_rs)
  t2 = _tc_b(s1, t1, dinv, b1_2d, W2)
  s2 = _sc_scatter(t2, src_rs, dst_rs)
  t3 = _tc_b(s2, t2, dinv, b2_2d, W3)
  s3 = _sc_scatter(t3, src_rs, dst_rs)

  h, att = _tc_c1(s3, t3, dinv, b3_2d, Wattn, Wv)
  att_rs = att.reshape(NG, 12)
  h_rs = h.reshape(NG, 12 * H)
  hf_rs, sc = _tc_c2(att_rs, h_rs)
  scores = sc.reshape(NG, 12, 1)

  hf = hf_rs.reshape(N, H)
  hf_pad = jnp.concatenate([hf, jnp.zeros((NPB - N, H), jnp.float32)])
  hf_flat = hf_pad.reshape(NPB * H)

  maxp, sump = _sc_pool(hf_flat, batch_rows)
  maxp3 = maxp.reshape(NW, GPB, H)
  sump3 = sump.reshape(NW, GPB, H)

  out = _tc_d(maxp3, sump3, cnt_t, Wlin, blin2d)
  return (out, scores)


# speedup vs baseline: 1.0300x; 1.0300x over previous
import functools

import jax
import jax.numpy as jnp
from jax import lax
from jax.experimental import pallas as pl
from jax.experimental.pallas import tpu as pltpu
from jax.experimental.pallas import tpu_sc as plsc

N = 9996
E = 319872
NG = 833
DIN = 128
H = 64

NC = 2
NS = 16
NW = NC * NS
L = 16

EPW = 10240
EP = EPW * NW
NCH = 80
CH = 128
AR = 10240
NPB = 10240
GPB = 848
GF = GPB * H

_HI = jax.lax.Precision.HIGHEST


def _iota16():
  return lax.iota(jnp.int32, L)


def _sc_hist_body(dst_rs, batch_rs, deg_out, cnt_out, idx, ones_v, bidx, zb,
                  degacc, cntacc):
  s = lax.axis_index("s")
  c = lax.axis_index("c")
  w = s * NC + c

  one = jnp.ones((L,), jnp.int32)
  zero = jnp.zeros((L,), jnp.int32)
  for j in range(CH // L):
    ones_v[pl.ds(j * L, L)] = one
  for j in range(1024 // L):
    zb[pl.ds(j * L, L)] = zero

  pltpu.sync_copy(zb.at[pl.ds(0, 640)], degacc.at[pl.ds(s * 640, 640)])

  @pl.when(s == 0)
  def _():
    pltpu.sync_copy(zb.at[pl.ds(0, 848)], cntacc)

  plsc.subcore_barrier()

  pltpu.sync_copy(dst_rs.at[w], idx)

  def _chunk(jo, _):
    for ji in range(4):
      pltpu.sync_copy(ones_v, degacc.at[idx.at[jo * 4 + ji]], add=True)
    return 0

  lax.fori_loop(0, NCH // 4, _chunk, 0)

  pltpu.sync_copy(batch_rs.at[w], bidx)
  for j in range(4):
    pltpu.sync_copy(ones_v.at[pl.ds(0, 80)], cntacc.at[bidx.at[j]], add=True)

  plsc.subcore_barrier()

  pltpu.sync_copy(degacc.at[pl.ds(s * 640, 640)],
                  deg_out.at[c, pl.ds(s * 640, 640)])

  @pl.when(s == 0)
  def _():
    pltpu.sync_copy(cntacc, cnt_out.at[c])


def _sc_hist(dst_rs, batch_rs):
  f = pl.kernel(
      _sc_hist_body,
      out_type=(
          jax.ShapeDtypeStruct((NC, AR), jnp.int32),
          jax.ShapeDtypeStruct((NC, GPB), jnp.int32),
      ),
      mesh=plsc.VectorSubcoreMesh(core_axis_name="c", subcore_axis_name="s",
                                  num_cores=NC, num_subcores=NS),
      scratch_types=[
          pltpu.VMEM((NCH, CH), jnp.int32),
          pltpu.VMEM((CH,), jnp.int32),
          pltpu.VMEM((4, 80), jnp.int32),
          pltpu.VMEM((1024,), jnp.int32),
          pltpu.VMEM_SHARED((AR,), jnp.int32),
          pltpu.VMEM_SHARED((GPB,), jnp.int32),
      ],
  )
  return f(dst_rs, batch_rs)


def _sc_scatter_body(t_hbm, src_rs, dst_rs, out_hbm, idxs, idxd, rows, rows2,
                     rows3, rows4, rows5, rows6, rows7, rows8, zb, acc, sem,
                     sem2):
  s = lax.axis_index("s")
  c = lax.axis_index("c")
  w = s * NC + c

  zrow = jnp.zeros((L,), jnp.float32)
  for r in range(64):
    for k in range(H // L):
      zb[r, pl.ds(k * L, L)] = zrow

  for i in range(10):
    pltpu.sync_copy(zb, acc.at[pl.ds(s * 640 + i * 64, 64)])

  plsc.subcore_barrier()

  pltpu.sync_copy(src_rs.at[w], idxs)
  pltpu.sync_copy(dst_rs.at[w], idxd)

  bufs = (rows, rows2, rows3, rows4, rows5, rows6, rows7, rows8)
  for j in range(7):
    pltpu.make_async_copy(t_hbm.at[idxs.at[j]], bufs[j], sem).start()

  def _chunk(jo, _):
    for ji in range(8):
      j = jo * 8 + ji
      bufa = bufs[ji]
      bufb = bufs[(ji + 7) % 8]
      pltpu.make_async_copy(
          t_hbm.at[idxs.at[jnp.minimum(j, NCH - 1)]], bufa, sem).wait()

      @pl.when(j >= 1)
      def _():
        pltpu.make_async_copy(
            bufb, acc.at[idxd.at[jnp.maximum(j - 1, 0)]], sem2).wait()

      @pl.when(j < NCH - 7)
      def _():
        pltpu.make_async_copy(
            t_hbm.at[idxs.at[jnp.minimum(j + 7, NCH - 1)]], bufb, sem).start()

      pltpu.make_async_copy(bufa, acc.at[idxd.at[j]], sem2).start(add=True)
    return 0

  lax.fori_loop(0, NCH // 8, _chunk, 0)
  pltpu.make_async_copy(bufs[7], acc.at[idxd.at[NCH - 1]], sem2).wait()

  plsc.subcore_barrier()

  pltpu.sync_copy(acc.at[pl.ds(s * 640, 640)],
                  out_hbm.at[c, pl.ds(s * 640, 640)])


def _sc_scatter(t, src_rs, dst_rs):
  f = pl.kernel(
      _sc_scatter_body,
      out_type=jax.ShapeDtypeStruct((NC, AR, H), jnp.float32),
      mesh=plsc.VectorSubcoreMesh(core_axis_name="c", subcore_axis_name="s",
                                  num_cores=NC, num_subcores=NS),
      compiler_params=pltpu.CompilerParams(use_tc_tiling_on_sc=False),
      scratch_types=[
          pltpu.VMEM((NCH, CH), jnp.int32),
          pltpu.VMEM((NCH, CH), jnp.int32),
          pltpu.VMEM((CH, H), jnp.float32),
          pltpu.VMEM((CH, H), jnp.float32),
          pltpu.VMEM((CH, H), jnp.float32),
          pltpu.VMEM((CH, H), jnp.float32),
          pltpu.VMEM((CH, H), jnp.float32),
          pltpu.VMEM((CH, H), jnp.float32),
          pltpu.VMEM((CH, H), jnp.float32),
          pltpu.VMEM((CH, H), jnp.float32),
          pltpu.VMEM((64, H), jnp.float32),
          pltpu.VMEM_SHARED((AR, H), jnp.float32),
          pltpu.SemaphoreType.DMA,
          pltpu.SemaphoreType.DMA,
      ],
  )
  return f(t, src_rs, dst_rs)


def _sc_pool_body(hf_flat, batch_rs, max_out, sum_out, rowsf, bsm, lmax, lsum,
                  sem):
  s = lax.axis_index("s")
  c = lax.axis_index("c")
  w = s * NC + c

  zrow = jnp.zeros((L,), jnp.float32)

  def _zero(i, _):
    for k in range(8):
      lmax[pl.ds(i * (8 * L) + k * L, L)] = zrow
      lsum[pl.ds(i * (8 * L) + k * L, L)] = zrow
    return 0

  lax.fori_loop(0, GF // (8 * L), _zero, 0)

  pltpu.sync_copy(hf_flat.at[pl.ds(w * 320 * H, 320 * H)], rowsf)
  pltpu.sync_copy(batch_rs.at[w], bsm)

  it = _iota16()

  def _row(r, _):
    g = bsm[pl.ds(r, L)][0]
    base = g * H
    rbase = r * H
    for k in range(H // L):
      fidx = base + k * L + it
      v = plsc.load_gather(rowsf, [rbase + k * L + it])
      cur = plsc.load_gather(lmax, [fidx])
      plsc.store_scatter(lmax, [fidx], jnp.maximum(cur, v))
      plsc.addupdate_scatter(lsum, [fidx], v)
    return 0

  lax.fori_loop(0, 320, _row, 0)

  pltpu.sync_copy(lmax, max_out.at[w])
  pltpu.sync_copy(lsum, sum_out.at[w])


def _sc_pool(hf_flat, batch_rs):
  f = pl.kernel(
      _sc_pool_body,
      out_type=(
          jax.ShapeDtypeStruct((NW, GF), jnp.float32),
          jax.ShapeDtypeStruct((NW, GF), jnp.float32),
      ),
      mesh=plsc.VectorSubcoreMesh(core_axis_name="c", subcore_axis_name="s",
                                  num_cores=NC, num_subcores=NS),
      compiler_params=pltpu.CompilerParams(needs_layout_passes=False),
      scratch_types=[
          pltpu.VMEM((320 * H,), jnp.float32),
          pltpu.VMEM((384,), jnp.int32),
          pltpu.VMEM((GF,), jnp.float32),
          pltpu.VMEM((GF,), jnp.float32),
          pltpu.SemaphoreType.DMA,
      ],
  )
  return f(hf_flat, batch_rs)


def _tc_a_body(dp_t, x, w1, t1, dinv):
  deg = (dp_t[:, 0:1] + dp_t[:, 1:2] + 1).astype(jnp.float32)
  dv = lax.rsqrt(deg)
  xw = lax.dot_general(x[...], w1[...], (((1,), (1,)), ((), ())), precision=_HI,
                       preferred_element_type=jnp.float32)
  t1[...] = xw * dv[:N]
  dinv[...] = dv[:N]


def _tc_a(dp_t, x, w1):
  return pl.pallas_call(
      _tc_a_body,
      out_shape=(
          jax.ShapeDtypeStruct((N, H), jnp.float32),
          jax.ShapeDtypeStruct((N, 1), jnp.float32),
      ),
  )(dp_t, x, w1)


def _tc_b_body(sp, t, dinv, b, wn, tn):
  agg = (sp[0, :N] + sp[1, :N] + t[...]) * dinv[...] + b[...]
  h = jnp.maximum(agg, 0.0)
  hw = lax.dot_general(h, wn[...], (((1,), (1,)), ((), ())), precision=_HI,
                       preferred_element_type=jnp.float32)
  tn[...] = hw * dinv[...]


def _tc_b(sp, t, dinv, b2d, wn):
  return pl.pallas_call(
      _tc_b_body,
      out_shape=jax.ShapeDtypeStruct((N, H), jnp.float32),
  )(sp, t, dinv, b2d, wn)


def _tc_c1_body(sp, t, dinv, b, wattn, wv, h_out, att_out):
  agg = (sp[0, :N] + sp[1, :N] + t[...]) * dinv[...] + b[...]
  h = jnp.maximum(agg, 0.0)
  energy = jnp.tanh(lax.dot_general(h, wattn[...], (((1,), (1,)), ((), ())),
                                    precision=_HI,
                                    preferred_element_type=jnp.float32))
  att = lax.dot_general(energy, wv[...], (((1,), (1,)), ((), ())),
                        precision=_HI, preferred_element_type=jnp.float32)
  h_out[...] = h
  att_out[...] = att


def _tc_c1(sp, t, dinv, b2d, wattn, wv):
  return pl.pallas_call(
      _tc_c1_body,
      out_shape=(
          jax.ShapeDtypeStruct((N, H), jnp.float32),
          jax.ShapeDtypeStruct((N, 1), jnp.float32),
      ),
  )(sp, t, dinv, b2d, wattn, wv)


def _tc_c2_body(att, h_rs, hf_out, sc_out):
  a = att[...]
  m = jnp.max(a, axis=1, keepdims=True)
  ex = jnp.exp(a - m)
  sc = ex / jnp.sum(ex, axis=1, keepdims=True)
  rows = lax.broadcasted_iota(jnp.int32, (12, 12 * H), 0)
  cols = lax.broadcasted_iota(jnp.int32, (12, 12 * H), 1) // H
  rmat = (rows == cols).astype(jnp.float32)
  expd = lax.dot_general(sc, rmat, (((1,), (0,)), ((), ())), precision=_HI,
                         preferred_element_type=jnp.float32)
  hf_out[...] = h_rs[...] * expd
  sc_out[...] = sc


def _tc_c2(att_rs, h_rs):
  return pl.pallas_call(
      _tc_c2_body,
      out_shape=(
          jax.ShapeDtypeStruct((NG, 12 * H), jnp.float32),
          jax.ShapeDtypeStruct((NG, 12), jnp.float32),
      ),
  )(att_rs, h_rs)


def _tc_d_body(maxp, sump, cnt_t, wlin, blin, out):
  gmax = jnp.max(maxp[...], axis=0)[:NG]
  gsum = jnp.sum(sump[...], axis=0)[:NG]
  cnt = (cnt_t[:, 0:1] + cnt_t[:, 1:2]).astype(jnp.float32)[:NG]
  gmean = gsum / jnp.maximum(cnt, 1.0)
  pooled = jnp.concatenate([gmax, gmean], axis=1)
  ow = lax.dot_general(pooled, wlin[...], (((1,), (1,)), ((), ())),
                       precision=_HI, preferred_element_type=jnp.float32)
  out[...] = jnp.maximum(ow + blin[...], 0.0)


def _tc_d(maxp3, sump3, cnt_t, wlin, blin2d):
  return pl.pallas_call(
      _tc_d_body,
      out_shape=jax.ShapeDtypeStruct((NG, 32), jnp.float32),
  )(maxp3, sump3, cnt_t, wlin, blin2d)


def kernel(data_x, data_edge_index, data_batch, W1, b1, W2, b2, W3, b3, Wattn,
           Wv, Wlin, blin):
  src = data_edge_index[0].astype(jnp.int32)
  dst = data_edge_index[1].astype(jnp.int32)
  npad = EP - E
  pad_ar = lax.iota(jnp.int32, npad)
  src_pad = jnp.concatenate([src, (pad_ar * 1237) % N])
  dst_pad = jnp.concatenate([dst, N + pad_ar % (AR - N)])
  src_rs = src_pad.reshape(NW, NCH, CH)
  dst_rs = dst_pad.reshape(NW, NCH, CH)

  batch_pad = jnp.concatenate([
      data_batch.astype(jnp.int32),
      jnp.full((NPB - N,), NG, jnp.int32),
  ])
  batch_rs = batch_pad.reshape(NW, 4, 80)
  batch_rows = jnp.concatenate(
      [batch_pad.reshape(NW, 320), jnp.full((NW, 64), NG, jnp.int32)], axis=1)

  degp, cntp = _sc_hist(dst_rs, batch_rs)
  dp_t = degp.T
  cnt_t = cntp.T

  b1_2d = b1.reshape(1, H)
  b2_2d = b2.reshape(1, H)
  b3_2d = b3.reshape(1, H)
  blin2d = blin.reshape(1, 32)

  t1, dinv = _tc_a(dp_t, data_x, W1)
  s1 = _sc_scatter(t1, src_rs, dst_rs)
  t2 = _tc_b(s1, t1, dinv, b1_2d, W2)
  s2 = _sc_scatter(t2, src_rs, dst_rs)
  t3 = _tc_b(s2, t2, dinv, b2_2d, W3)
  s3 = _sc_scatter(t3, src_rs, dst_rs)

  h, att = _tc_c1(s3, t3, dinv, b3_2d, Wattn, Wv)
  att_rs = att.reshape(NG, 12)
  h_rs = h.reshape(NG, 12 * H)
  hf_rs, sc = _tc_c2(att_rs, h_rs)
  scores = sc.reshape(NG, 12, 1)

  hf = hf_rs.reshape(N, H)
  hf_pad = jnp.concatenate([hf, jnp.zeros((NPB - N, H), jnp.float32)])
  hf_flat = hf_pad.reshape(NPB * H)

  maxp, sump = _sc_pool(hf_flat, batch_rows)
  maxp3 = maxp.reshape(NW, GPB, H)
  sump3 = sump.reshape(NW, GPB, H)

  out = _tc_d(maxp3, sump3, cnt_t, Wlin, blin2d)
  return (out, scores)


# speedup vs baseline: 1.0301x; 1.0001x over previous
import jax
import jax.numpy as jnp
from jax import lax
from jax.experimental import pallas as pl
from jax.experimental.pallas import tpu as pltpu
from jax.experimental.pallas import tpu_sc as plsc

N = 9996
E = 319872
NG = 833
DIN = 128
H = 64

NC = 2
NS = 16
NW = NC * NS
L = 16

EPW = 10240
EP = EPW * NW
NCH = 80
CH = 128
AR = 10240
NPB = 10240
GPB = 848
GF = GPB * H

_HI = jax.lax.Precision.HIGHEST


def _iota16():
  return lax.iota(jnp.int32, L)


def _sc_hist_body(dst_rs, batch_rs, deg_out, cnt_out, idx, ones_v, bidx, zb,
                  degacc, cntacc):
  s = lax.axis_index("s")
  c = lax.axis_index("c")
  w = s * NC + c

  one = jnp.ones((L,), jnp.int32)
  zero = jnp.zeros((L,), jnp.int32)
  for j in range(CH // L):
    ones_v[pl.ds(j * L, L)] = one
  for j in range(1024 // L):
    zb[pl.ds(j * L, L)] = zero

  pltpu.sync_copy(zb.at[pl.ds(0, 640)], degacc.at[pl.ds(s * 640, 640)])

  @pl.when(s == 0)
  def _():
    pltpu.sync_copy(zb.at[pl.ds(0, 848)], cntacc)

  plsc.subcore_barrier()

  pltpu.sync_copy(dst_rs.at[w], idx)

  def _chunk(jo, _):
    for ji in range(4):
      pltpu.sync_copy(ones_v, degacc.at[idx.at[jo * 4 + ji]], add=True)
    return 0

  lax.fori_loop(0, NCH // 4, _chunk, 0)

  pltpu.sync_copy(batch_rs.at[w], bidx)
  for j in range(4):
    pltpu.sync_copy(ones_v.at[pl.ds(0, 80)], cntacc.at[bidx.at[j]], add=True)

  plsc.subcore_barrier()

  pltpu.sync_copy(degacc.at[pl.ds(s * 640, 640)],
                  deg_out.at[c, pl.ds(s * 640, 640)])

  @pl.when(s == 0)
  def _():
    pltpu.sync_copy(cntacc, cnt_out.at[c])


def _sc_hist(dst_rs, batch_rs):
  f = pl.kernel(
      _sc_hist_body,
      out_type=(
          jax.ShapeDtypeStruct((NC, AR), jnp.int32),
          jax.ShapeDtypeStruct((NC, GPB), jnp.int32),
      ),
      mesh=plsc.VectorSubcoreMesh(core_axis_name="c", subcore_axis_name="s",
                                  num_cores=NC, num_subcores=NS),
      scratch_types=[
          pltpu.VMEM((NCH, CH), jnp.int32),
          pltpu.VMEM((CH,), jnp.int32),
          pltpu.VMEM((4, 80), jnp.int32),
          pltpu.VMEM((1024,), jnp.int32),
          pltpu.VMEM_SHARED((AR,), jnp.int32),
          pltpu.VMEM_SHARED((GPB,), jnp.int32),
      ],
  )
  return f(dst_rs, batch_rs)


def _sc_scatter_body(t_hbm, src_rs, dst_rs, out_hbm, idxs, idxd, rows, rows2,
                     rows3, rows4, rows5, rows6, rows7, rows8, zb, acc, sem,
                     sem2):
  s = lax.axis_index("s")
  c = lax.axis_index("c")
  w = s * NC + c

  zrow = jnp.zeros((L,), jnp.float32)
  for r in range(64):
    for k in range(H // L):
      zb[r, pl.ds(k * L, L)] = zrow

  for i in range(10):
    pltpu.sync_copy(zb, acc.at[pl.ds(s * 640 + i * 64, 64)])

  plsc.subcore_barrier()

  pltpu.sync_copy(src_rs.at[w], idxs)
  pltpu.sync_copy(dst_rs.at[w], idxd)

  bufs = (rows, rows2, rows3, rows4, rows5, rows6, rows7, rows8)
  for j in range(7):
    pltpu.make_async_copy(t_hbm.at[idxs.at[j]], bufs[j], sem).start()

  def _chunk(jo, _):
    for ji in range(8):
      j = jo * 8 + ji
      bufa = bufs[ji]
      bufb = bufs[(ji + 7) % 8]
      pltpu.make_async_copy(
          t_hbm.at[idxs.at[jnp.minimum(j, NCH - 1)]], bufa, sem).wait()

      @pl.when(j >= 1)
      def _():
        pltpu.make_async_copy(
            bufb, acc.at[idxd.at[jnp.maximum(j - 1, 0)]], sem2).wait()

      @pl.when(j < NCH - 7)
      def _():
        pltpu.make_async_copy(
            t_hbm.at[idxs.at[jnp.minimum(j + 7, NCH - 1)]], bufb, sem).start()

      pltpu.make_async_copy(bufa, acc.at[idxd.at[j]], sem2).start(add=True)
    return 0

  lax.fori_loop(0, NCH // 8, _chunk, 0)
  pltpu.make_async_copy(bufs[7], acc.at[idxd.at[NCH - 1]], sem2).wait()

  plsc.subcore_barrier()

  pltpu.sync_copy(acc.at[pl.ds(s * 640, 640)],
                  out_hbm.at[c, pl.ds(s * 640, 640)])


def _sc_scatter(t, src_rs, dst_rs):
  f = pl.kernel(
      _sc_scatter_body,
      out_type=jax.ShapeDtypeStruct((NC, AR, H), jnp.float32),
      mesh=plsc.VectorSubcoreMesh(core_axis_name="c", subcore_axis_name="s",
                                  num_cores=NC, num_subcores=NS),
      compiler_params=pltpu.CompilerParams(use_tc_tiling_on_sc=False),
      scratch_types=[
          pltpu.VMEM((NCH, CH), jnp.int32),
          pltpu.VMEM((NCH, CH), jnp.int32),
          pltpu.VMEM((CH, H), jnp.float32),
          pltpu.VMEM((CH, H), jnp.float32),
          pltpu.VMEM((CH, H), jnp.float32),
          pltpu.VMEM((CH, H), jnp.float32),
          pltpu.VMEM((CH, H), jnp.float32),
          pltpu.VMEM((CH, H), jnp.float32),
          pltpu.VMEM((CH, H), jnp.float32),
          pltpu.VMEM((CH, H), jnp.float32),
          pltpu.VMEM((64, H), jnp.float32),
          pltpu.VMEM_SHARED((AR, H), jnp.float32),
          pltpu.SemaphoreType.DMA,
          pltpu.SemaphoreType.DMA,
      ],
  )
  return f(t, src_rs, dst_rs)


def _sc_pool_body(hf_flat, batch_rs, max_out, sum_out, rowsf, bsm, lmax, lsum,
                  sem):
  s = lax.axis_index("s")
  c = lax.axis_index("c")
  w = s * NC + c

  zrow = jnp.zeros((L,), jnp.float32)

  def _zero(i, _):
    for k in range(8):
      lmax[pl.ds(i * (8 * L) + k * L, L)] = zrow
      lsum[pl.ds(i * (8 * L) + k * L, L)] = zrow
    return 0

  lax.fori_loop(0, GF // (8 * L), _zero, 0)

  pltpu.sync_copy(hf_flat.at[pl.ds(w * 320 * H, 320 * H)], rowsf)
  pltpu.sync_copy(batch_rs.at[w], bsm)

  it = _iota16()

  def _row(r, _):
    g = bsm[pl.ds(r, L)][0]
    base = g * H
    rbase = r * H
    for k in range(H // L):
      fidx = base + k * L + it
      v = plsc.load_gather(rowsf, [rbase + k * L + it])
      cur = plsc.load_gather(lmax, [fidx])
      plsc.store_scatter(lmax, [fidx], jnp.maximum(cur, v))
      plsc.addupdate_scatter(lsum, [fidx], v)
    return 0

  lax.fori_loop(0, 320, _row, 0)

  pltpu.sync_copy(lmax, max_out.at[w])
  pltpu.sync_copy(lsum, sum_out.at[w])


def _sc_pool(hf_flat, batch_rs):
  f = pl.kernel(
      _sc_pool_body,
      out_type=(
          jax.ShapeDtypeStruct((NW, GF), jnp.float32),
          jax.ShapeDtypeStruct((NW, GF), jnp.float32),
      ),
      mesh=plsc.VectorSubcoreMesh(core_axis_name="c", subcore_axis_name="s",
                                  num_cores=NC, num_subcores=NS),
      compiler_params=pltpu.CompilerParams(needs_layout_passes=False),
      scratch_types=[
          pltpu.VMEM((320 * H,), jnp.float32),
          pltpu.VMEM((384,), jnp.int32),
          pltpu.VMEM((GF,), jnp.float32),
          pltpu.VMEM((GF,), jnp.float32),
          pltpu.SemaphoreType.DMA,
      ],
  )
  return f(hf_flat, batch_rs)


def _tc_a_body(dp_t, x, w1, t1, dinv):
  deg = (dp_t[:, 0:1] + dp_t[:, 1:2] + 1).astype(jnp.float32)
  dv = lax.rsqrt(deg)
  xw = lax.dot_general(x[...], w1[...], (((1,), (1,)), ((), ())), precision=_HI,
                       preferred_element_type=jnp.float32)
  t1[...] = xw * dv[:N]
  dinv[...] = dv[:N]


def _tc_a(dp_t, x, w1):
  return pl.pallas_call(
      _tc_a_body,
      out_shape=(
          jax.ShapeDtypeStruct((N, H), jnp.float32),
          jax.ShapeDtypeStruct((N, 1), jnp.float32),
      ),
  )(dp_t, x, w1)


def _tc_b_body(sp, t, dinv, b, wn, tn):
  agg = (sp[0, :N] + sp[1, :N] + t[...]) * dinv[...] + b[...]
  h = jnp.maximum(agg, 0.0)
  hw = lax.dot_general(h, wn[...], (((1,), (1,)), ((), ())), precision=_HI,
                       preferred_element_type=jnp.float32)
  tn[...] = hw * dinv[...]


def _tc_b(sp, t, dinv, b2d, wn):
  return pl.pallas_call(
      _tc_b_body,
      out_shape=jax.ShapeDtypeStruct((N, H), jnp.float32),
  )(sp, t, dinv, b2d, wn)


def _tc_c1_body(sp, t, dinv, b, wattn, wv, h_out, att_out):
  agg = (sp[0, :N] + sp[1, :N] + t[...]) * dinv[...] + b[...]
  h = jnp.maximum(agg, 0.0)
  energy = jnp.tanh(lax.dot_general(h, wattn[...], (((1,), (1,)), ((), ())),
                                    precision=_HI,
                                    preferred_element_type=jnp.float32))
  att = lax.dot_general(energy, wv[...], (((1,), (1,)), ((), ())),
                        precision=_HI, preferred_element_type=jnp.float32)
  h_out[...] = h
  att_out[...] = att


def _tc_c1(sp, t, dinv, b2d, wattn, wv):
  return pl.pallas_call(
      _tc_c1_body,
      out_shape=(
          jax.ShapeDtypeStruct((N, H), jnp.float32),
          jax.ShapeDtypeStruct((N, 1), jnp.float32),
      ),
  )(sp, t, dinv, b2d, wattn, wv)


def _tc_c2_body(att, h_rs, hf_out, sc_out):
  a = att[...]
  m = jnp.max(a, axis=1, keepdims=True)
  ex = jnp.exp(a - m)
  sc = ex / jnp.sum(ex, axis=1, keepdims=True)
  rows = lax.broadcasted_iota(jnp.int32, (12, 12 * H), 0)
  cols = lax.broadcasted_iota(jnp.int32, (12, 12 * H), 1) // H
  rmat = (rows == cols).astype(jnp.float32)
  expd = lax.dot_general(sc, rmat, (((1,), (0,)), ((), ())), precision=_HI,
                         preferred_element_type=jnp.float32)
  hf_out[...] = h_rs[...] * expd
  sc_out[...] = sc


def _tc_c2(att_rs, h_rs):
  return pl.pallas_call(
      _tc_c2_body,
      out_shape=(
          jax.ShapeDtypeStruct((NG, 12 * H), jnp.float32),
          jax.ShapeDtypeStruct((NG, 12), jnp.float32),
      ),
  )(att_rs, h_rs)


def _tc_d_body(maxp, sump, cnt_t, wlin, blin, out):
  gmax = jnp.max(maxp[...], axis=0)[:NG]
  gsum = jnp.sum(sump[...], axis=0)[:NG]
  cnt = (cnt_t[:, 0:1] + cnt_t[:, 1:2]).astype(jnp.float32)[:NG]
  gmean = gsum / jnp.maximum(cnt, 1.0)
  pooled = jnp.concatenate([gmax, gmean], axis=1)
  ow = lax.dot_general(pooled, wlin[...], (((1,), (1,)), ((), ())),
                       precision=_HI, preferred_element_type=jnp.float32)
  out[...] = jnp.maximum(ow + blin[...], 0.0)


def _tc_d(maxp3, sump3, cnt_t, wlin, blin2d):
  return pl.pallas_call(
      _tc_d_body,
      out_shape=jax.ShapeDtypeStruct((NG, 32), jnp.float32),
  )(maxp3, sump3, cnt_t, wlin, blin2d)


def kernel(data_x, data_edge_index, data_batch, W1, b1, W2, b2, W3, b3, Wattn,
           Wv, Wlin, blin):
  src = data_edge_index[0].astype(jnp.int32)
  dst = data_edge_index[1].astype(jnp.int32)
  npad = EP - E
  pad_ar = lax.iota(jnp.int32, npad)
  src_pad = jnp.concatenate([src, (pad_ar * 1237) % N])
  dst_pad = jnp.concatenate([dst, N + pad_ar % (AR - N)])
  src_rs = src_pad.reshape(NW, NCH, CH)
  dst_rs = dst_pad.reshape(NW, NCH, CH)

  batch_pad = jnp.concatenate([
      data_batch.astype(jnp.int32),
      jnp.full((NPB - N,), NG, jnp.int32),
  ])
  batch_rs = batch_pad.reshape(NW, 4, 80)
  batch_rows = jnp.concatenate(
      [batch_pad.reshape(NW, 320), jnp.full((NW, 64), NG, jnp.int32)], axis=1)

  degp, cntp = _sc_hist(dst_rs, batch_rs)
  dp_t = degp.T
  cnt_t = cntp.T

  b1_2d = b1.reshape(1, H)
  b2_2d = b2.reshape(1, H)
  b3_2d = b3.reshape(1, H)
  blin2d = blin.reshape(1, 32)

  t1, dinv = _tc_a(dp_t, data_x, W1)
  s1 = _sc_scatter(t1, src_rs, dst_rs)
  t2 = _tc_b(s1, t1, dinv, b1_2d, W2)
  s2 = _sc_scatter(t2, src_rs, dst_rs)
  t3 = _tc_b(s2, t2, dinv, b2_2d, W3)
  s3 = _sc_scatter(t3, src_rs, dst_rs)

  h, att = _tc_c1(s3, t3, dinv, b3_2d, Wattn, Wv)
  att_rs = att.reshape(NG, 12)
  h_rs = h.reshape(NG, 12 * H)
  hf_rs, sc = _tc_c2(att_rs, h_rs)
  scores = sc.reshape(NG, 12, 1)

  hf = hf_rs.reshape(N, H)
  hf_pad = jnp.concatenate([hf, jnp.zeros((NPB - N, H), jnp.float32)])
  hf_flat = hf_pad.reshape(NPB * H)

  maxp, sump = _sc_pool(hf_flat, batch_rows)
  maxp3 = maxp.reshape(NW, GPB, H)
  sump3 = sump.reshape(NW, GPB, H)

  out = _tc_d(maxp3, sump3, cnt_t, Wlin, blin2d)
  return (out, scores)
